# baseline (speedup 1.0000x reference)
import functools

import jax
import jax.numpy as jnp
from jax import lax
from jax.experimental import pallas as pl
from jax.experimental.pallas import tpu as pltpu
from jax.experimental.pallas import tpu_sc as plsc

_BATCH, _SEQ, _EMBED = 4096, 50, 64
_VOCAB = 1000000
_NC, _NS = 2, 16
_NW = _NC * _NS
_BPW = _BATCH // _NW
_LANES = 16
_COLS = _EMBED // _LANES
_INV_SEQ = 1.0 / _SEQ

_PACK_IN_COLS = 512
_PACK_OUT_ROWS = _PACK_IN_COLS // 2
_PACK_GRID = (_VOCAB + _PACK_IN_COLS - 1) // _PACK_IN_COLS


def _pack_body(emb_t_ref, out_ref):
  y = emb_t_ref[...].T
  out_ref[...] = jnp.concatenate([y, y], axis=1)


def _pack_table(emb_t):
  return pl.pallas_call(
      _pack_body,
      grid=(_PACK_GRID,),
      in_specs=[pl.BlockSpec((_EMBED, _PACK_IN_COLS), lambda i: (0, i))],
      out_specs=pl.BlockSpec((_PACK_IN_COLS, 2 * _EMBED), lambda i: (i, 0)),
      out_shape=jax.ShapeDtypeStruct((_VOCAB, 2 * _EMBED), jnp.float32),
  )(emb_t)


def _make_cbow():
  mesh = plsc.VectorSubcoreMesh(
      core_axis_name="c", subcore_axis_name="s",
      num_cores=_NC, num_subcores=_NS)

  @functools.partial(
      pl.kernel,
      mesh=mesh,
      compiler_params=pltpu.CompilerParams(use_tc_tiling_on_sc=True),
      out_type=jax.ShapeDtypeStruct((_BATCH // 2, 2 * _EMBED), jnp.float32),
      scratch_types=[
          pltpu.VMEM((_SEQ, _BPW), jnp.int32),
          pltpu.VMEM((_BPW, 2 * _EMBED), jnp.float32),
          pltpu.VMEM((_BPW, 2 * _EMBED), jnp.float32),
          pltpu.VMEM((_BPW // 2, 2 * _EMBED), jnp.float32),
          pltpu.SemaphoreType.DMA,
          pltpu.SemaphoreType.DMA,
      ],
  )
  def cbow(xt_hbm, tab_hbm, out_hbm, idx_v, rows0, rows1, acc, sem0, sem1):
    wid = lax.axis_index("s") * _NC + lax.axis_index("c")
    col0 = wid * _BPW

    pltpu.sync_copy(xt_hbm.at[:, pl.ds(col0, _BPW)], idx_v)

    rows = (rows0, rows1)
    sems = (sem0, sem1)

    pending = pltpu.async_copy(tab_hbm.at[idx_v.at[0]], rows0, sem0)

    for s in range(_SEQ):
      b = s & 1
      pending.wait()
      if s + 1 < _SEQ:
        pending = pltpu.async_copy(
            tab_hbm.at[idx_v.at[s + 1]], rows[1 - b], sems[1 - b])
      src = rows[b]

      if s == 0:
        def init_body(i, _):
          for d in range(2):
            r = i * 2 + d
            aoff = d * _EMBED
            for c in range(_COLS):
              acc[i, pl.ds(aoff + c * _LANES, _LANES)] = (
                  src[r, pl.ds(c * _LANES, _LANES)])
          return 0
        lax.fori_loop(0, _BPW // 2, init_body, 0)
      else:
        def acc_body(i, _, src=src):
          for d in range(2):
            r = i * 2 + d
            aoff = d * _EMBED
            for c in range(_COLS):
              plsc.addupdate(
                  acc.at[i, pl.ds(aoff + c * _LANES, _LANES)],
                  src[r, pl.ds(c * _LANES, _LANES)])
          return 0
        lax.fori_loop(0, _BPW // 2, acc_body, 0)

    def scale_body(i, _):
      for c in range(2 * _COLS):
        sl = pl.ds(c * _LANES, _LANES)
        acc[i, sl] = acc[i, sl] * _INV_SEQ
      return 0
    lax.fori_loop(0, _BPW // 2, scale_body, 0)

    pltpu.sync_copy(acc, out_hbm.at[pl.ds(wid * (_BPW // 2), _BPW // 2)])

  return cbow


_cbow = _make_cbow()


@jax.jit
def kernel(X, emb):
  xt = jnp.transpose(X.astype(jnp.int32))
  packed = _pack_table(jnp.transpose(emb))
  out2 = _cbow(xt, packed)
  return out2.reshape(_BATCH, _EMBED)


# speedup vs baseline: 4.0604x; 4.0604x over previous
import functools

import jax
import jax.numpy as jnp
from jax import lax
from jax.experimental import pallas as pl
from jax.experimental.pallas import tpu as pltpu
from jax.experimental.pallas import tpu_sc as plsc

_BATCH, _SEQ, _EMBED = 4096, 50, 64
_VOCAB = 1000000
_NC, _NS = 2, 16
_NW = _NC * _NS
_BPW = _BATCH // _NW
_LANES = 16
_COLS = _EMBED // _LANES
_UNROLL = 4
_INV_SEQ = 1.0 / _SEQ

_PACK_IN_COLS = 8192
_PACK_GRID = (_VOCAB + _PACK_IN_COLS - 1) // _PACK_IN_COLS


def _pack_body(emb_t_ref, out_ref):
  out_ref[:, 0:_EMBED] = emb_t_ref[...].T


def _pack_table(emb_t):
  return pl.pallas_call(
      _pack_body,
      grid=(_PACK_GRID,),
      in_specs=[pl.BlockSpec((_EMBED, _PACK_IN_COLS), lambda i: (0, i))],
      out_specs=pl.BlockSpec((_PACK_IN_COLS, 2 * _EMBED), lambda i: (i, 0)),
      out_shape=jax.ShapeDtypeStruct((_VOCAB, 2 * _EMBED), jnp.float32),
  )(emb_t)


def _make_cbow():
  mesh = plsc.VectorSubcoreMesh(
      core_axis_name="c", subcore_axis_name="s",
      num_cores=_NC, num_subcores=_NS)

  @functools.partial(
      pl.kernel,
      mesh=mesh,
      compiler_params=pltpu.CompilerParams(use_tc_tiling_on_sc=False),
      out_type=jax.ShapeDtypeStruct((_BATCH, _EMBED), jnp.float32),
      scratch_types=[
          pltpu.VMEM((_SEQ, _BPW), jnp.int32),
          pltpu.VMEM((_SEQ, _BPW), jnp.int32),
          pltpu.VMEM((_BPW, _EMBED), jnp.float32),
          pltpu.VMEM((_BPW, _EMBED), jnp.float32),
          pltpu.VMEM((_BPW, _EMBED), jnp.float32),
          pltpu.SemaphoreType.DMA,
          pltpu.SemaphoreType.DMA,
      ],
  )
  def cbow(xt_hbm, tab_hbm, out_hbm, idx_v, id2_v, rows0, rows1, acc,
           sem0, sem1):
    wid = lax.axis_index("s") * _NC + lax.axis_index("c")
    col0 = wid * _BPW

    pltpu.sync_copy(xt_hbm.at[:, pl.ds(col0, _BPW)], idx_v)

    def dbl_body(s, _):
      for c in range(_BPW // _LANES):
        sl = pl.ds(c * _LANES, _LANES)
        id2_v[s, sl] = jax.lax.shift_left(idx_v[s, sl], 1)
      return 0
    lax.fori_loop(0, _SEQ, dbl_body, 0)

    rows = (rows0, rows1)
    sems = (sem0, sem1)

    pending = pltpu.async_copy(tab_hbm.at[id2_v.at[0]], rows0, sem0)

    for s in range(_SEQ):
      b = s & 1
      pending.wait()
      if s + 1 < _SEQ:
        pending = pltpu.async_copy(
            tab_hbm.at[id2_v.at[s + 1]], rows[1 - b], sems[1 - b])
      src = rows[b]

      if s == 0:
        def init_body(i, _):
          r = i * _UNROLL
          for d in range(_UNROLL):
            for c in range(_COLS):
              acc[r + d, pl.ds(c * _LANES, _LANES)] = (
                  src[r + d, pl.ds(c * _LANES, _LANES)])
          return 0
        lax.fori_loop(0, _BPW // _UNROLL, init_body, 0)
      else:
        def acc_body(i, _, src=src):
          r = i * _UNROLL
          for d in range(_UNROLL):
            for c in range(_COLS):
              plsc.addupdate(
                  acc.at[r + d, pl.ds(c * _LANES, _LANES)],
                  src[r + d, pl.ds(c * _LANES, _LANES)])
          return 0
        lax.fori_loop(0, _BPW // _UNROLL, acc_body, 0)

    def scale_body(i, _):
      r = i * _UNROLL
      for d in range(_UNROLL):
        for c in range(_COLS):
          sl = pl.ds(c * _LANES, _LANES)
          acc[r + d, sl] = acc[r + d, sl] * _INV_SEQ
      return 0
    lax.fori_loop(0, _BPW // _UNROLL, scale_body, 0)

    pltpu.sync_copy(acc, out_hbm.at[pl.ds(col0, _BPW)])

  return cbow


_cbow = _make_cbow()


@jax.jit
def kernel(X, emb):
  xt = jnp.transpose(X.astype(jnp.int32))
  packed = _pack_table(jnp.transpose(emb))
  tab = packed.reshape(2 * _VOCAB, _EMBED)
  return _cbow(xt, tab)


# speedup vs baseline: 4.3020x; 1.0595x over previous
import functools

import jax
import jax.numpy as jnp
from jax import lax
from jax.experimental import pallas as pl
from jax.experimental.pallas import tpu as pltpu
from jax.experimental.pallas import tpu_sc as plsc

_BATCH, _SEQ, _EMBED = 4096, 50, 64
_VOCAB = 1000000
_NC, _NS = 2, 16
_NW = _NC * _NS
_BPW = _BATCH // _NW
_LANES = 16
_COLS = _EMBED // _LANES
_UNROLL = 4
_INV_SEQ = 1.0 / _SEQ

_PACK_IN_COLS = 16384
_PACK_GRID = (_VOCAB + _PACK_IN_COLS - 1) // _PACK_IN_COLS


def _pack_body(emb_t_ref, out_ref):
  out_ref[:, 0:_EMBED] = emb_t_ref[...].T


def _pack_table(emb_t):
  return pl.pallas_call(
      _pack_body,
      grid=(_PACK_GRID,),
      in_specs=[pl.BlockSpec((_EMBED, _PACK_IN_COLS), lambda i: (0, i))],
      out_specs=pl.BlockSpec((_PACK_IN_COLS, 2 * _EMBED), lambda i: (i, 0)),
      out_shape=jax.ShapeDtypeStruct((_VOCAB, 2 * _EMBED), jnp.float32),
  )(emb_t)


def _make_cbow():
  mesh = plsc.VectorSubcoreMesh(
      core_axis_name="c", subcore_axis_name="s",
      num_cores=_NC, num_subcores=_NS)

  @functools.partial(
      pl.kernel,
      mesh=mesh,
      compiler_params=pltpu.CompilerParams(use_tc_tiling_on_sc=False),
      out_type=jax.ShapeDtypeStruct((_BATCH, _EMBED), jnp.float32),
      scratch_types=[
          pltpu.VMEM((_SEQ, _BPW), jnp.int32),
          pltpu.VMEM((_SEQ, _BPW), jnp.int32),
          pltpu.VMEM((_BPW, _EMBED), jnp.float32),
          pltpu.VMEM((_BPW, _EMBED), jnp.float32),
          pltpu.VMEM((_BPW, _EMBED), jnp.float32),
          pltpu.SemaphoreType.DMA,
          pltpu.SemaphoreType.DMA,
      ],
  )
  def cbow(xt_hbm, tab_hbm, out_hbm, idx_v, id2_v, rows0, rows1, acc,
           sem0, sem1):
    wid = lax.axis_index("s") * _NC + lax.axis_index("c")
    col0 = wid * _BPW

    pltpu.sync_copy(xt_hbm.at[:, pl.ds(col0, _BPW)], idx_v)

    def dbl_body(s, _):
      for c in range(_BPW // _LANES):
        sl = pl.ds(c * _LANES, _LANES)
        id2_v[s, sl] = jax.lax.shift_left(idx_v[s, sl], 1)
      return 0
    lax.fori_loop(0, _SEQ, dbl_body, 0)

    rows = (rows0, rows1)
    sems = (sem0, sem1)

    pending = pltpu.async_copy(tab_hbm.at[id2_v.at[0]], rows0, sem0)

    for s in range(_SEQ):
      b = s & 1
      pending.wait()
      if s + 1 < _SEQ:
        pending = pltpu.async_copy(
            tab_hbm.at[id2_v.at[s + 1]], rows[1 - b], sems[1 - b])
      src = rows[b]

      if s == 0:
        def init_body(i, _):
          r = i * _UNROLL
          for d in range(_UNROLL):
            for c in range(_COLS):
              acc[r + d, pl.ds(c * _LANES, _LANES)] = (
                  src[r + d, pl.ds(c * _LANES, _LANES)])
          return 0
        lax.fori_loop(0, _BPW // _UNROLL, init_body, 0)
      else:
        def acc_body(i, _, src=src):
          r = i * _UNROLL
          for d in range(_UNROLL):
            for c in range(_COLS):
              plsc.addupdate(
                  acc.at[r + d, pl.ds(c * _LANES, _LANES)],
                  src[r + d, pl.ds(c * _LANES, _LANES)])
          return 0
        lax.fori_loop(0, _BPW // _UNROLL, acc_body, 0)

    def scale_body(i, _):
      r = i * _UNROLL
      for d in range(_UNROLL):
        for c in range(_COLS):
          sl = pl.ds(c * _LANES, _LANES)
          acc[r + d, sl] = acc[r + d, sl] * _INV_SEQ
      return 0
    lax.fori_loop(0, _BPW // _UNROLL, scale_body, 0)

    pltpu.sync_copy(acc, out_hbm.at[pl.ds(col0, _BPW)])

  return cbow


_cbow = _make_cbow()


@jax.jit
def kernel(X, emb):
  xt = jnp.transpose(X.astype(jnp.int32))
  packed = _pack_table(jnp.transpose(emb))
  tab = packed.reshape(2 * _VOCAB, _EMBED)
  return _cbow(xt, tab)


# speedup vs baseline: 4.3816x; 1.0185x over previous
import functools

import jax
import jax.numpy as jnp
from jax import lax
from jax.experimental import pallas as pl
from jax.experimental.pallas import tpu as pltpu
from jax.experimental.pallas import tpu_sc as plsc

_BATCH, _SEQ, _EMBED = 4096, 50, 64
_VOCAB = 1000000
_NC, _NS = 2, 16
_NW = _NC * _NS
_BPW = _BATCH // _NW
_LANES = 16
_COLS = _EMBED // _LANES
_UNROLL = 4
_INV_SEQ = 1.0 / _SEQ

_PACK_IN_COLS = 32768
_PACK_GRID = (_VOCAB + _PACK_IN_COLS - 1) // _PACK_IN_COLS


def _pack_body(emb_t_ref, out_ref):
  out_ref[:, 0:_EMBED] = emb_t_ref[...].T


def _pack_table(emb_t):
  return pl.pallas_call(
      _pack_body,
      grid=(_PACK_GRID,),
      in_specs=[pl.BlockSpec((_EMBED, _PACK_IN_COLS), lambda i: (0, i))],
      out_specs=pl.BlockSpec((_PACK_IN_COLS, 2 * _EMBED), lambda i: (i, 0)),
      out_shape=jax.ShapeDtypeStruct((_VOCAB, 2 * _EMBED), jnp.float32),
  )(emb_t)


def _make_cbow():
  mesh = plsc.VectorSubcoreMesh(
      core_axis_name="c", subcore_axis_name="s",
      num_cores=_NC, num_subcores=_NS)

  @functools.partial(
      pl.kernel,
      mesh=mesh,
      compiler_params=pltpu.CompilerParams(use_tc_tiling_on_sc=False),
      out_type=jax.ShapeDtypeStruct((_BATCH, _EMBED), jnp.float32),
      scratch_types=[
          pltpu.VMEM((_SEQ, _BPW), jnp.int32),
          pltpu.VMEM((_SEQ, _BPW), jnp.int32),
          pltpu.VMEM((_BPW, _EMBED), jnp.float32),
          pltpu.VMEM((_BPW, _EMBED), jnp.float32),
          pltpu.VMEM((_BPW, _EMBED), jnp.float32),
          pltpu.SemaphoreType.DMA,
          pltpu.SemaphoreType.DMA,
      ],
  )
  def cbow(xt_hbm, tab_hbm, out_hbm, idx_v, id2_v, rows0, rows1, acc,
           sem0, sem1):
    wid = lax.axis_index("s") * _NC + lax.axis_index("c")
    col0 = wid * _BPW

    pltpu.sync_copy(xt_hbm.at[:, pl.ds(col0, _BPW)], idx_v)

    def dbl_body(s, _):
      for c in range(_BPW // _LANES):
        sl = pl.ds(c * _LANES, _LANES)
        id2_v[s, sl] = jax.lax.shift_left(idx_v[s, sl], 1)
      return 0
    lax.fori_loop(0, _SEQ, dbl_body, 0)

    rows = (rows0, rows1)
    sems = (sem0, sem1)

    pending = pltpu.async_copy(tab_hbm.at[id2_v.at[0]], rows0, sem0)

    for s in range(_SEQ):
      b = s & 1
      pending.wait()
      if s + 1 < _SEQ:
        pending = pltpu.async_copy(
            tab_hbm.at[id2_v.at[s + 1]], rows[1 - b], sems[1 - b])
      src = rows[b]

      if s == 0:
        def init_body(i, _):
          r = i * _UNROLL
          for d in range(_UNROLL):
            for c in range(_COLS):
              acc[r + d, pl.ds(c * _LANES, _LANES)] = (
                  src[r + d, pl.ds(c * _LANES, _LANES)])
          return 0
        lax.fori_loop(0, _BPW // _UNROLL, init_body, 0)
      else:
        def acc_body(i, _, src=src):
          r = i * _UNROLL
          for d in range(_UNROLL):
            for c in range(_COLS):
              plsc.addupdate(
                  acc.at[r + d, pl.ds(c * _LANES, _LANES)],
                  src[r + d, pl.ds(c * _LANES, _LANES)])
          return 0
        lax.fori_loop(0, _BPW // _UNROLL, acc_body, 0)

    def scale_body(i, _):
      r = i * _UNROLL
      for d in range(_UNROLL):
        for c in range(_COLS):
          sl = pl.ds(c * _LANES, _LANES)
          acc[r + d, sl] = acc[r + d, sl] * _INV_SEQ
      return 0
    lax.fori_loop(0, _BPW // _UNROLL, scale_body, 0)

    pltpu.sync_copy(acc, out_hbm.at[pl.ds(col0, _BPW)])

  return cbow


_cbow = _make_cbow()


@jax.jit
def kernel(X, emb):
  xt = jnp.transpose(X.astype(jnp.int32))
  packed = _pack_table(jnp.transpose(emb))
  tab = packed.reshape(2 * _VOCAB, _EMBED)
  return _cbow(xt, tab)
